# baseline (speedup 1.0000x reference)
import functools

import jax
import jax.numpy as jnp
from jax import lax
from jax.experimental import pallas as pl
from jax.experimental.pallas import tpu as pltpu
from jax.experimental.pallas import tpu_sc as plsc

N = 10000
E = 320000
D = 128
G = 64

NC = 2
NS = 16
NW = NC * NS
K = 112
NCH = 90
EPW = NCH * K
N_ACC = N + 8
SEGS = (16, 16, 16, 16, 16, 10)
SEGMAX = 16
ZCH = 80
DEG_PAD = 10240
DWPT = DEG_PAD // NS

BN = 1000
NBLK = N // BN


def _mesh():
    return plsc.VectorSubcoreMesh(
        core_axis_name="c", subcore_axis_name="s", num_cores=NC, num_subcores=NS
    )


@functools.partial(
    pl.kernel,
    out_type=jax.ShapeDtypeStruct((NC, 1, DEG_PAD), jnp.float32),
    mesh=_mesh(),
    scratch_types=[
        pltpu.VMEM((NCH, K), jnp.int32),
        pltpu.VMEM((K,), jnp.float32),
        pltpu.VMEM((DWPT,), jnp.float32),
        pltpu.VMEM_SHARED((DEG_PAD,), jnp.float32),
        pltpu.SemaphoreType.DMA,
    ],
)
def _sc_deg(dst_hbm, out_hbm, di_v, ones_v, buf_v, acc_sh, sem):
    c = lax.axis_index("c")
    s = lax.axis_index("s")
    wid = s * NC + c

    def fill_ones(i, carry):
        ones_v[pl.ds(i * 16, 16)] = jnp.ones((16,), jnp.float32)
        return carry

    lax.fori_loop(0, K // 16, fill_ones, 0)

    def fill_zero(i, carry):
        buf_v[pl.ds(i * 16, 16)] = jnp.zeros((16,), jnp.float32)
        return carry

    lax.fori_loop(0, DWPT // 16, fill_zero, 0)
    pltpu.sync_copy(buf_v, acc_sh.at[pl.ds(s * DWPT, DWPT)])
    pltpu.sync_copy(dst_hbm.at[wid], di_v)
    plsc.subcore_barrier()

    FG = 5

    def group(t, carry):
        j0 = t * FG
        for i in range(FG):
            pltpu.make_async_copy(ones_v, acc_sh.at[di_v.at[j0 + i]], sem).start(
                add=True
            )
        for i in range(FG):
            pltpu.make_async_copy(ones_v, acc_sh.at[di_v.at[j0 + i]], sem).wait()
        return carry

    lax.fori_loop(0, NCH // FG, group, 0)
    plsc.subcore_barrier()

    pltpu.sync_copy(acc_sh.at[pl.ds(s * DWPT, DWPT)], buf_v)
    pltpu.sync_copy(buf_v, out_hbm.at[c, 0, pl.ds(s * DWPT, DWPT)])


@functools.partial(
    pl.kernel,
    out_type=jax.ShapeDtypeStruct((NC, N, D), jnp.float32),
    mesh=_mesh(),
    scratch_types=[
        pltpu.VMEM((SEGMAX, K), jnp.int32),
        pltpu.VMEM((SEGMAX, K), jnp.int32),
        pltpu.VMEM((K, D), jnp.float32),
        pltpu.VMEM((K, D), jnp.float32),
        pltpu.VMEM((K, D), jnp.float32),
        pltpu.VMEM_SHARED((N_ACC, D), jnp.float32),
        pltpu.SemaphoreType.DMA,
        pltpu.SemaphoreType.DMA,
        pltpu.SemaphoreType.DMA,
        pltpu.SemaphoreType.DMA,
        pltpu.SemaphoreType.DMA,
        pltpu.SemaphoreType.DMA,
    ],
)
def _sc_agg(h_hbm, src_hbm, dst_hbm, out_hbm, si_v, di_v, rb0, rb1, rb2,
            acc_sh, g0, g1, g2, s0, s1, s2):
    rows = (rb0, rb1, rb2)
    gsem = (g0, g1, g2)
    ssem = (s0, s1, s2)
    c = lax.axis_index("c")
    s = lax.axis_index("s")
    wid = s * NC + c

    def zrow(i, carry):
        def zcol(j, inner):
            rb0[i, pl.ds(j * 16, 16)] = jnp.zeros((16,), jnp.float32)
            return inner

        return lax.fori_loop(0, D // 16, zcol, carry)

    lax.fori_loop(0, K, zrow, 0)

    r0 = s * 640
    nz = jnp.where(s == NS - 1, 5, 8)

    def zcopy(z, carry):
        pltpu.sync_copy(rb0.at[pl.ds(0, ZCH)], acc_sh.at[pl.ds(r0 + z * ZCH, ZCH)])
        return carry

    lax.fori_loop(0, nz, zcopy, 0)

    def g_desc(j, b):
        return pltpu.make_async_copy(h_hbm.at[si_v.at[j]], rows[b], gsem[b])

    def s_desc(j, b):
        return pltpu.make_async_copy(rows[b], acc_sh.at[di_v.at[j]], ssem[b])

    def step(j, b):
        g_desc(j, b).wait()
        s_desc(j, b).start(add=True)

    def run_seg(cnt):
        g_desc(0, 0).start()
        g_desc(1, 1).start()
        step(0, 0)
        g_desc(2, 2).start()
        nt = (cnt - 4) // 3

        def group(t, carry):
            for i in range(3):
                j = 1 + 3 * t + i
                b = (1 + i) % 3
                step(j, b)
                s_desc(j - 1, (b + 2) % 3).wait()
                g_desc(j + 2, (b + 2) % 3).start()
            return carry

        lax.fori_loop(0, nt, group, 0)
        for j in range(3 * nt + 1, cnt):
            step(j, j % 3)
            s_desc(j - 1, (j - 1) % 3).wait()
            if j + 2 <= cnt - 1:
                g_desc(j + 2, (j + 2) % 3).start()
        s_desc(cnt - 1, (cnt - 1) % 3).wait()

    off = 0
    for gi, cnt in enumerate(SEGS):
        pltpu.sync_copy(src_hbm.at[wid, pl.ds(off, cnt)], si_v.at[pl.ds(0, cnt)])
        pltpu.sync_copy(dst_hbm.at[wid, pl.ds(off, cnt)], di_v.at[pl.ds(0, cnt)])
        if gi == 0:
            plsc.subcore_barrier()
        run_seg(cnt)
        off += cnt
    plsc.subcore_barrier()

    def wcopy(z, carry):
        pltpu.sync_copy(acc_sh.at[pl.ds(r0 + z * ZCH, ZCH)], rb0.at[pl.ds(0, ZCH)])
        pltpu.sync_copy(rb0.at[pl.ds(0, ZCH)], out_hbm.at[c, pl.ds(r0 + z * ZCH, ZCH)])
        return carry

    lax.fori_loop(0, nz, wcopy, 0)


def _tc_h1p(x, degT, W1):
    def body(x_ref, dg_ref, w_ref, hp_ref, dinv_ref):
        d = dg_ref[:, 0:1] + dg_ref[:, 1:2] + 1.0
        dinv = lax.rsqrt(d)
        mm = jnp.dot(x_ref[...], w_ref[...], preferred_element_type=jnp.float32)
        hp_ref[...] = dinv * mm
        dinv_ref[...] = dinv

    return pl.pallas_call(
        body,
        grid=(NBLK,),
        in_specs=[
            pl.BlockSpec((BN, D), lambda i: (i, 0)),
            pl.BlockSpec((BN, 2), lambda i: (i, 0)),
            pl.BlockSpec((D, D), lambda i: (0, 0)),
        ],
        out_specs=[
            pl.BlockSpec((BN, D), lambda i: (i, 0)),
            pl.BlockSpec((BN, 1), lambda i: (i, 0)),
        ],
        out_shape=[
            jax.ShapeDtypeStruct((N, D), jnp.float32),
            jax.ShapeDtypeStruct((N, 1), jnp.float32),
        ],
    )(x, degT, W1)


def _tc_mid(acc0, acc1, h1p, dinv, b1r, W2):
    def body(a0, a1, hp, dv, br, w_ref, out_ref):
        h1 = jnp.maximum(dv[...] * (a0[...] + a1[...] + hp[...]) + br[...], 0.0)
        out_ref[...] = dv[...] * jnp.dot(
            h1, w_ref[...], preferred_element_type=jnp.float32
        )

    return pl.pallas_call(
        body,
        grid=(NBLK,),
        in_specs=[
            pl.BlockSpec((BN, D), lambda i: (i, 0)),
            pl.BlockSpec((BN, D), lambda i: (i, 0)),
            pl.BlockSpec((BN, D), lambda i: (i, 0)),
            pl.BlockSpec((BN, 1), lambda i: (i, 0)),
            pl.BlockSpec((1, D), lambda i: (0, 0)),
            pl.BlockSpec((D, D), lambda i: (0, 0)),
        ],
        out_specs=pl.BlockSpec((BN, D), lambda i: (i, 0)),
        out_shape=jax.ShapeDtypeStruct((N, D), jnp.float32),
    )(acc0, acc1, h1p, dinv, b1r, W2)


def _tc_final(acc0, acc1, h2p, dinv, b2r, batch_col, W_fc, bfc_r):
    def body(a0, a1, hp, dv, br, bt, wf, bf, out_ref, sums, cnts):
        i = pl.program_id(0)

        @pl.when(i == 0)
        def _init():
            sums[...] = jnp.zeros((G, D), jnp.float32)
            cnts[...] = jnp.zeros((G, D), jnp.float32)

        h2 = jnp.maximum(dv[...] * (a0[...] + a1[...] + hp[...]) + br[...], 0.0)
        gids = lax.broadcasted_iota(jnp.int32, (BN, G), 1)
        onehot = (gids == bt[...]).astype(jnp.float32)
        dims = (((0,), (0,)), ((), ()))
        sums[...] += lax.dot_general(
            onehot, h2, dims, preferred_element_type=jnp.float32
        )
        cnts[...] += lax.dot_general(
            onehot, jnp.ones((BN, D), jnp.float32), dims,
            preferred_element_type=jnp.float32,
        )

        @pl.when(i == NBLK - 1)
        def _fin():
            pooled = sums[...] / jnp.maximum(cnts[...], 1.0)
            out_ref[...] = (
                jnp.dot(pooled, wf[...], preferred_element_type=jnp.float32) + bf[...]
            )

    return pl.pallas_call(
        body,
        grid=(NBLK,),
        in_specs=[
            pl.BlockSpec((BN, D), lambda i: (i, 0)),
            pl.BlockSpec((BN, D), lambda i: (i, 0)),
            pl.BlockSpec((BN, D), lambda i: (i, 0)),
            pl.BlockSpec((BN, 1), lambda i: (i, 0)),
            pl.BlockSpec((1, D), lambda i: (0, 0)),
            pl.BlockSpec((BN, 1), lambda i: (i, 0)),
            pl.BlockSpec((D, D), lambda i: (0, 0)),
            pl.BlockSpec((1, D), lambda i: (0, 0)),
        ],
        out_specs=pl.BlockSpec((G, D), lambda i: (0, 0)),
        out_shape=jax.ShapeDtypeStruct((G, D), jnp.float32),
        scratch_shapes=[
            pltpu.VMEM((G, D), jnp.float32),
            pltpu.VMEM((G, D), jnp.float32),
        ],
    )(acc0, acc1, h2p, dinv, b2r, batch_col, W_fc, bfc_r)


@jax.jit
def _impl(x, edge_index, batch, W1, b1, W2, b2, W_fc, b_fc):
    pad_w = EPW - E // NW
    src3 = jnp.pad(edge_index[0].reshape(NW, E // NW), ((0, 0), (0, pad_w)))
    src3 = src3.reshape(NW, NCH, K)
    trash = (N + (jnp.arange(pad_w, dtype=jnp.int32) % 8))[None, :]
    dst3 = jnp.concatenate(
        [edge_index[1].reshape(NW, E // NW),
         jnp.broadcast_to(trash, (NW, pad_w))], axis=1
    ).reshape(NW, NCH, K)

    deg2 = _sc_deg(dst3).reshape(NC, DEG_PAD)
    degT = jnp.transpose(deg2)[:N]

    h1p, dinv = _tc_h1p(x, degT, W1)
    acc1 = _sc_agg(h1p, src3, dst3)
    h2p = _tc_mid(acc1[0], acc1[1], h1p, dinv, b1.reshape(1, D), W2)
    acc2 = _sc_agg(h2p, src3, dst3)
    out = _tc_final(
        acc2[0], acc2[1], h2p, dinv,
        b2.reshape(1, D), batch.reshape(N, 1), W_fc, b_fc.reshape(1, D),
    )
    return out


def kernel(x, edge_index, batch, W1, b1, W2, b2, W_fc, b_fc):
    return _impl(x, edge_index, batch, W1, b1, W2, b2, W_fc, b_fc)


# speedup vs baseline: 1.6220x; 1.6220x over previous
import functools

import jax
import jax.numpy as jnp
from jax import lax
from jax.experimental import pallas as pl
from jax.experimental.pallas import tpu as pltpu
from jax.experimental.pallas import tpu_sc as plsc

N = 10000
E = 320000
D = 128
G = 64

NC = 2
NS = 16
NW = NC * NS
EPW = E // NW
K = 80
NCH = EPW // K
SEGS = (32, 32, 32, 29)
SEGMAX = 32
DEG_PAD = 10240
DWPT = DEG_PAD // NS

BN = 1000
NBLK = N // BN


def _mesh():
    return plsc.VectorSubcoreMesh(
        core_axis_name="c", subcore_axis_name="s", num_cores=NC, num_subcores=NS
    )


@functools.partial(
    pl.kernel,
    out_type=jax.ShapeDtypeStruct((NC, 1, DEG_PAD), jnp.float32),
    mesh=_mesh(),
    scratch_types=[
        pltpu.VMEM((NCH, K), jnp.int32),
        pltpu.VMEM((K,), jnp.float32),
        pltpu.VMEM((DWPT,), jnp.float32),
        pltpu.VMEM_SHARED((DEG_PAD,), jnp.float32),
        pltpu.SemaphoreType.DMA,
    ],
)
def _sc_deg(dst_hbm, out_hbm, di_v, ones_v, buf_v, acc_sh, sem):
    c = lax.axis_index("c")
    s = lax.axis_index("s")
    wid = s * NC + c

    def fill_ones(i, carry):
        ones_v[pl.ds(i * 16, 16)] = jnp.ones((16,), jnp.float32)
        return carry

    lax.fori_loop(0, K // 16, fill_ones, 0)

    def fill_zero(i, carry):
        buf_v[pl.ds(i * 16, 16)] = jnp.zeros((16,), jnp.float32)
        return carry

    lax.fori_loop(0, DWPT // 16, fill_zero, 0)
    pltpu.sync_copy(buf_v, acc_sh.at[pl.ds(s * DWPT, DWPT)])
    pltpu.sync_copy(dst_hbm.at[wid], di_v)
    plsc.subcore_barrier()

    FG = 5

    def group(t, carry):
        j0 = t * FG
        for i in range(FG):
            pltpu.make_async_copy(ones_v, acc_sh.at[di_v.at[j0 + i]], sem).start(
                add=True
            )
        for i in range(FG):
            pltpu.make_async_copy(ones_v, acc_sh.at[di_v.at[j0 + i]], sem).wait()
        return carry

    lax.fori_loop(0, NCH // FG, group, 0)
    plsc.subcore_barrier()

    pltpu.sync_copy(acc_sh.at[pl.ds(s * DWPT, DWPT)], buf_v)
    pltpu.sync_copy(buf_v, out_hbm.at[c, 0, pl.ds(s * DWPT, DWPT)])


@functools.partial(
    pl.kernel,
    out_type=jax.ShapeDtypeStruct((NC, N, D), jnp.float32),
    mesh=_mesh(),
    scratch_types=[
        pltpu.VMEM((SEGMAX, K), jnp.int32),
        pltpu.VMEM((SEGMAX, K), jnp.int32),
        pltpu.VMEM((K, D), jnp.float32),
        pltpu.VMEM((K, D), jnp.float32),
        pltpu.VMEM((K, D), jnp.float32),
        pltpu.VMEM_SHARED((N, D), jnp.float32),
        pltpu.SemaphoreType.DMA,
        pltpu.SemaphoreType.DMA,
        pltpu.SemaphoreType.DMA,
        pltpu.SemaphoreType.DMA,
        pltpu.SemaphoreType.DMA,
        pltpu.SemaphoreType.DMA,
    ],
)
def _sc_agg(h_hbm, src_hbm, dst_hbm, out_hbm, si_v, di_v, rb0, rb1, rb2,
            acc_sh, g0, g1, g2, s0, s1, s2):
    rows = (rb0, rb1, rb2)
    gsem = (g0, g1, g2)
    ssem = (s0, s1, s2)
    c = lax.axis_index("c")
    s = lax.axis_index("s")
    wid = s * NC + c

    def zrow(i, carry):
        def zcol(j, inner):
            rb0[i, pl.ds(j * 16, 16)] = jnp.zeros((16,), jnp.float32)
            return inner

        return lax.fori_loop(0, D // 16, zcol, carry)

    lax.fori_loop(0, K, zrow, 0)

    r0 = s * 640
    nz = jnp.where(s == NS - 1, 5, 8)

    def zcopy(z, carry):
        pltpu.sync_copy(rb0, acc_sh.at[pl.ds(r0 + z * K, K)])
        return carry

    lax.fori_loop(0, nz, zcopy, 0)

    def g_desc(j, b):
        return pltpu.make_async_copy(h_hbm.at[si_v.at[j]], rows[b], gsem[b])

    def s_desc(j, b):
        return pltpu.make_async_copy(rows[b], acc_sh.at[di_v.at[j]], ssem[b])

    def step(j, b):
        g_desc(j, b).wait()
        s_desc(j, b).start(add=True)

    def run_seg(cnt):
        g_desc(0, 0).start()
        g_desc(1, 1).start()
        step(0, 0)
        g_desc(2, 2).start()
        nt = (cnt - 4) // 3

        def group(t, carry):
            for i in range(3):
                j = 1 + 3 * t + i
                b = (1 + i) % 3
                step(j, b)
                s_desc(j - 1, (b + 2) % 3).wait()
                g_desc(j + 2, (b + 2) % 3).start()
            return carry

        lax.fori_loop(0, nt, group, 0)
        for j in range(3 * nt + 1, cnt):
            step(j, j % 3)
            s_desc(j - 1, (j - 1) % 3).wait()
            if j + 2 <= cnt - 1:
                g_desc(j + 2, (j + 2) % 3).start()
        s_desc(cnt - 1, (cnt - 1) % 3).wait()

    off = 0
    for gi, cnt in enumerate(SEGS):
        pltpu.sync_copy(src_hbm.at[wid, pl.ds(off, cnt)], si_v.at[pl.ds(0, cnt)])
        pltpu.sync_copy(dst_hbm.at[wid, pl.ds(off, cnt)], di_v.at[pl.ds(0, cnt)])
        if gi == 0:
            plsc.subcore_barrier()
        run_seg(cnt)
        off += cnt
    plsc.subcore_barrier()

    def wcopy(z, carry):
        pltpu.sync_copy(acc_sh.at[pl.ds(r0 + z * K, K)], rb0)
        pltpu.sync_copy(rb0, out_hbm.at[c, pl.ds(r0 + z * K, K)])
        return carry

    lax.fori_loop(0, nz, wcopy, 0)


def _tc_mm(x, W1):
    def body(x_ref, w_ref, out_ref):
        out_ref[...] = jnp.dot(
            x_ref[...], w_ref[...], preferred_element_type=jnp.float32
        )

    return pl.pallas_call(
        body,
        grid=(NBLK,),
        in_specs=[
            pl.BlockSpec((BN, D), lambda i: (i, 0)),
            pl.BlockSpec((D, D), lambda i: (0, 0)),
        ],
        out_specs=pl.BlockSpec((BN, D), lambda i: (i, 0)),
        out_shape=jax.ShapeDtypeStruct((N, D), jnp.float32),
    )(x, W1)


def _tc_h1p(xw, degT):
    def body(xw_ref, dg_ref, hp_ref, dinv_ref):
        d = dg_ref[:, 0:1] + dg_ref[:, 1:2] + 1.0
        dinv = lax.rsqrt(d)
        hp_ref[...] = dinv * xw_ref[...]
        dinv_ref[...] = dinv

    return pl.pallas_call(
        body,
        grid=(NBLK,),
        in_specs=[
            pl.BlockSpec((BN, D), lambda i: (i, 0)),
            pl.BlockSpec((BN, 2), lambda i: (i, 0)),
        ],
        out_specs=[
            pl.BlockSpec((BN, D), lambda i: (i, 0)),
            pl.BlockSpec((BN, 1), lambda i: (i, 0)),
        ],
        out_shape=[
            jax.ShapeDtypeStruct((N, D), jnp.float32),
            jax.ShapeDtypeStruct((N, 1), jnp.float32),
        ],
    )(xw, degT)


def _tc_mid(acc0, acc1, h1p, dinv, b1r, W2):
    def body(a0, a1, hp, dv, br, w_ref, out_ref):
        h1 = jnp.maximum(dv[...] * (a0[...] + a1[...] + hp[...]) + br[...], 0.0)
        out_ref[...] = dv[...] * jnp.dot(
            h1, w_ref[...], preferred_element_type=jnp.float32
        )

    return pl.pallas_call(
        body,
        grid=(NBLK,),
        in_specs=[
            pl.BlockSpec((BN, D), lambda i: (i, 0)),
            pl.BlockSpec((BN, D), lambda i: (i, 0)),
            pl.BlockSpec((BN, D), lambda i: (i, 0)),
            pl.BlockSpec((BN, 1), lambda i: (i, 0)),
            pl.BlockSpec((1, D), lambda i: (0, 0)),
            pl.BlockSpec((D, D), lambda i: (0, 0)),
        ],
        out_specs=pl.BlockSpec((BN, D), lambda i: (i, 0)),
        out_shape=jax.ShapeDtypeStruct((N, D), jnp.float32),
    )(acc0, acc1, h1p, dinv, b1r, W2)


def _tc_final(acc0, acc1, h2p, dinv, b2r, batch_col, W_fc, bfc_r):
    def body(a0, a1, hp, dv, br, bt, wf, bf, out_ref, sums, cnts):
        i = pl.program_id(0)

        @pl.when(i == 0)
        def _init():
            sums[...] = jnp.zeros((G, D), jnp.float32)
            cnts[...] = jnp.zeros((G, D), jnp.float32)

        h2 = jnp.maximum(dv[...] * (a0[...] + a1[...] + hp[...]) + br[...], 0.0)
        gids = lax.broadcasted_iota(jnp.int32, (BN, G), 1)
        onehot = (gids == bt[...]).astype(jnp.float32)
        dims = (((0,), (0,)), ((), ()))
        sums[...] += lax.dot_general(
            onehot, h2, dims, preferred_element_type=jnp.float32
        )
        cnts[...] += lax.dot_general(
            onehot, jnp.ones((BN, D), jnp.float32), dims,
            preferred_element_type=jnp.float32,
        )

        @pl.when(i == NBLK - 1)
        def _fin():
            pooled = sums[...] / jnp.maximum(cnts[...], 1.0)
            out_ref[...] = (
                jnp.dot(pooled, wf[...], preferred_element_type=jnp.float32) + bf[...]
            )

    return pl.pallas_call(
        body,
        grid=(NBLK,),
        in_specs=[
            pl.BlockSpec((BN, D), lambda i: (i, 0)),
            pl.BlockSpec((BN, D), lambda i: (i, 0)),
            pl.BlockSpec((BN, D), lambda i: (i, 0)),
            pl.BlockSpec((BN, 1), lambda i: (i, 0)),
            pl.BlockSpec((1, D), lambda i: (0, 0)),
            pl.BlockSpec((BN, 1), lambda i: (i, 0)),
            pl.BlockSpec((D, D), lambda i: (0, 0)),
            pl.BlockSpec((1, D), lambda i: (0, 0)),
        ],
        out_specs=pl.BlockSpec((G, D), lambda i: (0, 0)),
        out_shape=jax.ShapeDtypeStruct((G, D), jnp.float32),
        scratch_shapes=[
            pltpu.VMEM((G, D), jnp.float32),
            pltpu.VMEM((G, D), jnp.float32),
        ],
    )(acc0, acc1, h2p, dinv, b2r, batch_col, W_fc, bfc_r)


@jax.jit
def _impl(x, edge_index, batch, W1, b1, W2, b2, W_fc, b_fc):
    src3 = edge_index[0].reshape(NW, NCH, K)
    dst3 = edge_index[1].reshape(NW, NCH, K)

    deg2 = _sc_deg(dst3).reshape(NC, DEG_PAD)
    xw = _tc_mm(x, W1)
    degT = jnp.transpose(deg2)[:N]

    h1p, dinv = _tc_h1p(xw, degT)
    acc1 = _sc_agg(h1p, src3, dst3)
    h2p = _tc_mid(acc1[0], acc1[1], h1p, dinv, b1.reshape(1, D), W2)
    acc2 = _sc_agg(h2p, src3, dst3)
    out = _tc_final(
        acc2[0], acc2[1], h2p, dinv,
        b2.reshape(1, D), batch.reshape(N, 1), W_fc, b_fc.reshape(1, D),
    )
    return out


def kernel(x, edge_index, batch, W1, b1, W2, b2, W_fc, b_fc):
    return _impl(x, edge_index, batch, W1, b1, W2, b2, W_fc, b_fc)


# speedup vs baseline: 1.7005x; 1.0484x over previous
import functools

import jax
import jax.numpy as jnp
from jax import lax
from jax.experimental import pallas as pl
from jax.experimental.pallas import tpu as pltpu
from jax.experimental.pallas import tpu_sc as plsc

N = 10000
E = 320000
D = 128
G = 64

NC = 2
NS = 16
NW = NC * NS
EPW = E // NW
K = 80
NCH = EPW // K
SEGS = (64, 61)
SEGMAX = 64
DEG_PAD = 10240
DWPT = DEG_PAD // NS

BN = 1000
NBLK = N // BN


def _mesh():
    return plsc.VectorSubcoreMesh(
        core_axis_name="c", subcore_axis_name="s", num_cores=NC, num_subcores=NS
    )


@functools.partial(
    pl.kernel,
    out_type=jax.ShapeDtypeStruct((NC, 1, DEG_PAD), jnp.float32),
    mesh=_mesh(),
    scratch_types=[
        pltpu.VMEM((NCH, K), jnp.int32),
        pltpu.VMEM((K,), jnp.float32),
        pltpu.VMEM((DWPT,), jnp.float32),
        pltpu.VMEM_SHARED((DEG_PAD,), jnp.float32),
        pltpu.SemaphoreType.DMA,
    ],
)
def _sc_deg(dst_hbm, out_hbm, di_v, ones_v, buf_v, acc_sh, sem):
    c = lax.axis_index("c")
    s = lax.axis_index("s")
    wid = s * NC + c

    def fill_ones(i, carry):
        ones_v[pl.ds(i * 16, 16)] = jnp.ones((16,), jnp.float32)
        return carry

    lax.fori_loop(0, K // 16, fill_ones, 0)

    def fill_zero(i, carry):
        buf_v[pl.ds(i * 16, 16)] = jnp.zeros((16,), jnp.float32)
        return carry

    lax.fori_loop(0, DWPT // 16, fill_zero, 0)
    pltpu.sync_copy(buf_v, acc_sh.at[pl.ds(s * DWPT, DWPT)])
    pltpu.sync_copy(dst_hbm.at[wid], di_v)
    plsc.subcore_barrier()

    FG = 5

    def group(t, carry):
        j0 = t * FG
        for i in range(FG):
            pltpu.make_async_copy(ones_v, acc_sh.at[di_v.at[j0 + i]], sem).start(
                add=True
            )
        for i in range(FG):
            pltpu.make_async_copy(ones_v, acc_sh.at[di_v.at[j0 + i]], sem).wait()
        return carry

    lax.fori_loop(0, NCH // FG, group, 0)
    plsc.subcore_barrier()

    pltpu.sync_copy(acc_sh.at[pl.ds(s * DWPT, DWPT)], buf_v)
    pltpu.sync_copy(buf_v, out_hbm.at[c, 0, pl.ds(s * DWPT, DWPT)])


@functools.partial(
    pl.kernel,
    out_type=jax.ShapeDtypeStruct((NC, N, D), jnp.float32),
    mesh=_mesh(),
    scratch_types=[
        pltpu.VMEM((SEGMAX, K), jnp.int32),
        pltpu.VMEM((SEGMAX, K), jnp.int32),
        pltpu.VMEM((K, D), jnp.float32),
        pltpu.VMEM((K, D), jnp.float32),
        pltpu.VMEM((K, D), jnp.float32),
        pltpu.VMEM_SHARED((N, D), jnp.float32),
        pltpu.SemaphoreType.DMA,
        pltpu.SemaphoreType.DMA,
        pltpu.SemaphoreType.DMA,
        pltpu.SemaphoreType.DMA,
        pltpu.SemaphoreType.DMA,
        pltpu.SemaphoreType.DMA,
    ],
)
def _sc_agg(h_hbm, src_hbm, dst_hbm, out_hbm, si_v, di_v, rb0, rb1, rb2,
            acc_sh, g0, g1, g2, s0, s1, s2):
    rows = (rb0, rb1, rb2)
    gsem = (g0, g1, g2)
    ssem = (s0, s1, s2)
    c = lax.axis_index("c")
    s = lax.axis_index("s")
    wid = s * NC + c

    def zrow(i, carry):
        def zcol(j, inner):
            rb0[i, pl.ds(j * 16, 16)] = jnp.zeros((16,), jnp.float32)
            return inner

        return lax.fori_loop(0, D // 16, zcol, carry)

    lax.fori_loop(0, K, zrow, 0)

    r0 = s * 640
    nz = jnp.where(s == NS - 1, 5, 8)

    def zcopy(z, carry):
        pltpu.sync_copy(rb0, acc_sh.at[pl.ds(r0 + z * K, K)])
        return carry

    lax.fori_loop(0, nz, zcopy, 0)

    def g_desc(j, b):
        return pltpu.make_async_copy(h_hbm.at[si_v.at[j]], rows[b], gsem[b])

    def s_desc(j, b):
        return pltpu.make_async_copy(rows[b], acc_sh.at[di_v.at[j]], ssem[b])

    def step(j, b):
        g_desc(j, b).wait()
        s_desc(j, b).start(add=True)

    def run_seg(cnt):
        g_desc(0, 0).start()
        g_desc(1, 1).start()
        step(0, 0)
        g_desc(2, 2).start()
        nt = (cnt - 4) // 3

        def group(t, carry):
            for i in range(3):
                j = 1 + 3 * t + i
                b = (1 + i) % 3
                step(j, b)
                s_desc(j - 1, (b + 2) % 3).wait()
                g_desc(j + 2, (b + 2) % 3).start()
            return carry

        lax.fori_loop(0, nt, group, 0)
        for j in range(3 * nt + 1, cnt):
            step(j, j % 3)
            s_desc(j - 1, (j - 1) % 3).wait()
            if j + 2 <= cnt - 1:
                g_desc(j + 2, (j + 2) % 3).start()
        s_desc(cnt - 1, (cnt - 1) % 3).wait()

    off = 0
    for gi, cnt in enumerate(SEGS):
        pltpu.sync_copy(src_hbm.at[wid, pl.ds(off, cnt)], si_v.at[pl.ds(0, cnt)])
        pltpu.sync_copy(dst_hbm.at[wid, pl.ds(off, cnt)], di_v.at[pl.ds(0, cnt)])
        if gi == 0:
            plsc.subcore_barrier()
        run_seg(cnt)
        off += cnt
    plsc.subcore_barrier()

    def wcopy(z, carry):
        pltpu.sync_copy(acc_sh.at[pl.ds(r0 + z * K, K)], rb0)
        pltpu.sync_copy(rb0, out_hbm.at[c, pl.ds(r0 + z * K, K)])
        return carry

    lax.fori_loop(0, nz, wcopy, 0)


def _tc_h1p(x, degT, W1):
    def body(x_ref, dg_ref, w_ref, hp_ref, dinv_ref):
        d = dg_ref[:, 0:1] + dg_ref[:, 1:2] + 1.0
        dinv = lax.rsqrt(d)
        mm = jnp.dot(x_ref[...], w_ref[...], preferred_element_type=jnp.float32)
        hp_ref[...] = dinv * mm
        dinv_ref[...] = dinv

    return pl.pallas_call(
        body,
        grid=(NBLK,),
        in_specs=[
            pl.BlockSpec((BN, D), lambda i: (i, 0)),
            pl.BlockSpec((BN, 2), lambda i: (i, 0)),
            pl.BlockSpec((D, D), lambda i: (0, 0)),
        ],
        out_specs=[
            pl.BlockSpec((BN, D), lambda i: (i, 0)),
            pl.BlockSpec((BN, 1), lambda i: (i, 0)),
        ],
        out_shape=[
            jax.ShapeDtypeStruct((N, D), jnp.float32),
            jax.ShapeDtypeStruct((N, 1), jnp.float32),
        ],
    )(x, degT, W1)


def _tc_mid(acc0, acc1, h1p, dinv, b1r, W2):
    def body(a0, a1, hp, dv, br, w_ref, out_ref):
        h1 = jnp.maximum(dv[...] * (a0[...] + a1[...] + hp[...]) + br[...], 0.0)
        out_ref[...] = dv[...] * jnp.dot(
            h1, w_ref[...], preferred_element_type=jnp.float32
        )

    return pl.pallas_call(
        body,
        grid=(NBLK,),
        in_specs=[
            pl.BlockSpec((BN, D), lambda i: (i, 0)),
            pl.BlockSpec((BN, D), lambda i: (i, 0)),
            pl.BlockSpec((BN, D), lambda i: (i, 0)),
            pl.BlockSpec((BN, 1), lambda i: (i, 0)),
            pl.BlockSpec((1, D), lambda i: (0, 0)),
            pl.BlockSpec((D, D), lambda i: (0, 0)),
        ],
        out_specs=pl.BlockSpec((BN, D), lambda i: (i, 0)),
        out_shape=jax.ShapeDtypeStruct((N, D), jnp.float32),
    )(acc0, acc1, h1p, dinv, b1r, W2)


def _tc_final(acc0, acc1, h2p, dinv, b2r, batch_col, W_fc, bfc_r):
    def body(a0, a1, hp, dv, br, bt, wf, bf, out_ref, sums, cnts):
        i = pl.program_id(0)

        @pl.when(i == 0)
        def _init():
            sums[...] = jnp.zeros((G, D), jnp.float32)
            cnts[...] = jnp.zeros((G, D), jnp.float32)

        h2 = jnp.maximum(dv[...] * (a0[...] + a1[...] + hp[...]) + br[...], 0.0)
        gids = lax.broadcasted_iota(jnp.int32, (BN, G), 1)
        onehot = (gids == bt[...]).astype(jnp.float32)
        dims = (((0,), (0,)), ((), ()))
        sums[...] += lax.dot_general(
            onehot, h2, dims, preferred_element_type=jnp.float32
        )
        cnts[...] += lax.dot_general(
            onehot, jnp.ones((BN, D), jnp.float32), dims,
            preferred_element_type=jnp.float32,
        )

        @pl.when(i == NBLK - 1)
        def _fin():
            pooled = sums[...] / jnp.maximum(cnts[...], 1.0)
            out_ref[...] = (
                jnp.dot(pooled, wf[...], preferred_element_type=jnp.float32) + bf[...]
            )

    return pl.pallas_call(
        body,
        grid=(NBLK,),
        in_specs=[
            pl.BlockSpec((BN, D), lambda i: (i, 0)),
            pl.BlockSpec((BN, D), lambda i: (i, 0)),
            pl.BlockSpec((BN, D), lambda i: (i, 0)),
            pl.BlockSpec((BN, 1), lambda i: (i, 0)),
            pl.BlockSpec((1, D), lambda i: (0, 0)),
            pl.BlockSpec((BN, 1), lambda i: (i, 0)),
            pl.BlockSpec((D, D), lambda i: (0, 0)),
            pl.BlockSpec((1, D), lambda i: (0, 0)),
        ],
        out_specs=pl.BlockSpec((G, D), lambda i: (0, 0)),
        out_shape=jax.ShapeDtypeStruct((G, D), jnp.float32),
        scratch_shapes=[
            pltpu.VMEM((G, D), jnp.float32),
            pltpu.VMEM((G, D), jnp.float32),
        ],
    )(acc0, acc1, h2p, dinv, b2r, batch_col, W_fc, bfc_r)


@jax.jit
def _impl(x, edge_index, batch, W1, b1, W2, b2, W_fc, b_fc):
    src3 = edge_index[0].reshape(NW, NCH, K)
    dst3 = edge_index[1].reshape(NW, NCH, K)

    deg2 = _sc_deg(dst3).reshape(NC, DEG_PAD)
    degT = jnp.transpose(deg2)[:N]

    h1p, dinv = _tc_h1p(x, degT, W1)
    acc1 = _sc_agg(h1p, src3, dst3)
    h2p = _tc_mid(acc1[0], acc1[1], h1p, dinv, b1.reshape(1, D), W2)
    acc2 = _sc_agg(h2p, src3, dst3)
    out = _tc_final(
        acc2[0], acc2[1], h2p, dinv,
        b2.reshape(1, D), batch.reshape(N, 1), W_fc, b_fc.reshape(1, D),
    )
    return out


def kernel(x, edge_index, batch, W1, b1, W2, b2, W_fc, b_fc):
    return _impl(x, edge_index, batch, W1, b1, W2, b2, W_fc, b_fc)
